# baseline (speedup 1.0000x reference)
import functools

import jax
import jax.numpy as jnp
from jax.experimental import pallas as pl

_E = 64
_TOP_K = 8
_SCALE = 2.5


def _topk_from_logits(logits):
    iota = jax.lax.broadcasted_iota(jnp.int32, logits.shape, 0)
    work = logits
    idx_rows = []
    val_rows = []
    for k in range(_TOP_K):
        mk = jnp.max(work, axis=0, keepdims=True)
        if k == 0:
            m = mk
            denom = jnp.sum(jnp.exp(logits - m), axis=0, keepdims=True)
            inv = _SCALE / denom
        sel = jnp.min(jnp.where(work == mk, iota, _E), axis=0, keepdims=True)
        idx_rows.append(sel)
        val_rows.append(jnp.exp(mk - m) * inv)
        work = jnp.where(iota == sel, -jnp.inf, work)
    return jnp.concatenate(idx_rows, axis=0), jnp.concatenate(val_rows, axis=0)


def _router_block(w_ref, xa_ref, xb_ref, idxa_ref, vala_ref, idxb_ref, valb_ref):
    w = w_ref[...]
    for x_ref, idx_ref, val_ref in (
        (xa_ref, idxa_ref, vala_ref),
        (xb_ref, idxb_ref, valb_ref),
    ):
        logits = jax.lax.dot_general(
            w, x_ref[...],
            dimension_numbers=(((1,), (1,)), ((), ())),
            preferred_element_type=jnp.float32,
        )
        idx, val = _topk_from_logits(logits)
        idx_ref[...] = idx
        val_ref[...] = val


@functools.partial(jax.jit, static_argnames=("m_blk",))
def _router(flat, weight, m_blk):
    m_total, h = flat.shape
    half = m_total // 2
    n_steps = half // m_blk
    out_block = pl.BlockSpec((_TOP_K, m_blk), lambda i: (0, i))
    idx_shape = jax.ShapeDtypeStruct((_TOP_K, half), jnp.int32)
    val_shape = jax.ShapeDtypeStruct((_TOP_K, half), jnp.float32)
    idx_a, val_a, idx_b, val_b = pl.pallas_call(
        _router_block,
        grid=(n_steps,),
        in_specs=[
            pl.BlockSpec((_E, h), lambda i: (0, 0)),
            pl.BlockSpec((m_blk, h), lambda i: (i, 0)),
            pl.BlockSpec((m_blk, h), lambda i, n=n_steps: (n + i, 0)),
        ],
        out_specs=[out_block, out_block, out_block, out_block],
        out_shape=[idx_shape, val_shape, idx_shape, val_shape],
    )(weight, flat, flat)
    idx = jnp.concatenate([idx_a, idx_b], axis=1).T
    val = jnp.concatenate([val_a, val_b], axis=1).T
    return idx, val


def kernel(x, weight):
    Bx, Sx, Hx = x.shape
    flat = x.reshape(-1, Hx)
    idx, w = _router(flat, weight, 256)
    return idx.reshape(Bx, Sx, _TOP_K), w.reshape(Bx, Sx, _TOP_K)


# speedup vs baseline: 1.0794x; 1.0794x over previous
import functools

import jax
import jax.numpy as jnp
from jax.experimental import pallas as pl

_E = 64
_TOP_K = 8
_SCALE = 2.5


def _topk_from_logits(logits):
    iota = jax.lax.broadcasted_iota(jnp.int32, logits.shape, 0)
    work = logits
    idx_rows = []
    val_rows = []
    for k in range(_TOP_K):
        mk = jnp.max(work, axis=0, keepdims=True)
        if k == 0:
            m = mk
            denom = jnp.sum(jnp.exp(logits - m), axis=0, keepdims=True)
            inv = _SCALE / denom
        sel = jnp.min(jnp.where(work == mk, iota, _E), axis=0, keepdims=True)
        idx_rows.append(sel)
        val_rows.append(jnp.exp(mk - m) * inv)
        work = jnp.where(iota == sel, -jnp.inf, work)
    return jnp.concatenate(idx_rows, axis=0), jnp.concatenate(val_rows, axis=0)


def _router_block(w_ref, xa_ref, xb_ref, idx_ref, val_ref):
    h2 = xa_ref.shape[1]
    dn = (((1,), (1,)), ((), ()))
    logits = jax.lax.dot_general(
        w_ref[:, :h2], xa_ref[...], dimension_numbers=dn,
        preferred_element_type=jnp.float32,
    ) + jax.lax.dot_general(
        w_ref[:, h2:], xb_ref[...], dimension_numbers=dn,
        preferred_element_type=jnp.float32,
    )
    idx, val = _topk_from_logits(logits)
    idx_ref[...] = idx
    val_ref[...] = val


@functools.partial(jax.jit, static_argnames=("m_blk",))
def _router(flat, weight, m_blk):
    m_total, h = flat.shape
    n_steps = m_total // m_blk
    idx_t, val_t = pl.pallas_call(
        _router_block,
        grid=(n_steps,),
        in_specs=[
            pl.BlockSpec((_E, h), lambda i: (0, 0)),
            pl.BlockSpec((m_blk, h // 2), lambda i: (i, 0)),
            pl.BlockSpec((m_blk, h // 2), lambda i: (i, 1)),
        ],
        out_specs=[
            pl.BlockSpec((_TOP_K, m_blk), lambda i: (0, i)),
            pl.BlockSpec((_TOP_K, m_blk), lambda i: (0, i)),
        ],
        out_shape=[
            jax.ShapeDtypeStruct((_TOP_K, m_total), jnp.int32),
            jax.ShapeDtypeStruct((_TOP_K, m_total), jnp.float32),
        ],
    )(weight, flat, flat)
    return idx_t.T, val_t.T


def kernel(x, weight):
    Bx, Sx, Hx = x.shape
    flat = x.reshape(-1, Hx)
    idx, w = _router(flat, weight, 1024)
    return idx.reshape(Bx, Sx, _TOP_K), w.reshape(Bx, Sx, _TOP_K)
